# baseline (speedup 1.0000x reference)
import jax
import jax.numpy as jnp
from jax.experimental import pallas as pl
from jax.experimental.pallas import tpu as pltpu

_EMB = 64
_ROW_TILE = 512
_HEAD_W = 128
_OUT_COLS = 14 + 56 + 56


def _ceil_to(n, m):
    return ((n + m - 1) // m) * m


def _tile_spec(tile, cols):
    return pl.BlockSpec((tile, cols), lambda i: (i, 0))


def _full_spec(arr):
    return pl.BlockSpec(arr.shape, lambda i: (0,) * arr.ndim)


def _embed_and_project(x, w1, b1, w2, b2, projs, tile=_ROW_TILE):
    n = x.shape[0]
    nproj = len(projs)
    has_bias = [pb is not None for (_, pb) in projs]
    args = [x, w1, b1, w2, b2]
    specs = [_tile_spec(tile, x.shape[1]), _full_spec(w1), _full_spec(b1),
             _full_spec(w2), _full_spec(b2)]
    for (pw, pb) in projs:
        args.append(pw)
        specs.append(_full_spec(pw))
        if pb is not None:
            args.append(pb)
            specs.append(_full_spec(pb))

    def body(x_ref, w1_ref, b1_ref, w2_ref, b2_ref, *rest):
        nin = sum(1 + int(hb) for hb in has_bias)
        in_it = iter(rest[:nin])
        outs = rest[nin:]
        h = jnp.tanh(jnp.dot(x_ref[...], w1_ref[...],
                             preferred_element_type=jnp.float32) + b1_ref[...])
        emb = jnp.tanh(jnp.dot(h, w2_ref[...],
                               preferred_element_type=jnp.float32) + b2_ref[...])
        outs[0][...] = emb
        for k in range(nproj):
            wref = next(in_it)
            p = jnp.dot(emb, wref[...], preferred_element_type=jnp.float32)
            if has_bias[k]:
                p = p + next(in_it)[...]
            outs[1 + k][...] = p

    out_shapes = tuple(jax.ShapeDtypeStruct((n, _EMB), jnp.float32)
                       for _ in range(1 + nproj))
    out_specs = tuple(_tile_spec(tile, _EMB) for _ in range(1 + nproj))
    return pl.pallas_call(
        body,
        out_shape=out_shapes,
        grid=(n // tile,),
        in_specs=specs,
        out_specs=out_specs,
        compiler_params=pltpu.CompilerParams(
            dimension_semantics=("parallel",)),
    )(*args)


def _conv_out_call(aggs, cnt, own, wf, bf, wo1a, wo1b, bo1, wo2, bo2,
                   wnext=None, tile=_ROW_TILE):
    nagg = len(aggs)
    n = aggs[0].shape[0]

    def body(*refs):
        agg_refs = refs[:nagg]
        (cnt_ref, own_ref, wf_ref, bf_ref, wo1a_ref, wo1b_ref,
         bo1_ref, wo2_ref, bo2_ref) = refs[nagg:nagg + 9]
        rest = refs[nagg + 9:]
        araw = agg_refs[0][...]
        for k in range(1, nagg):
            araw = araw + agg_refs[k][...]
        a = jnp.dot(araw, wf_ref[...],
                    preferred_element_type=jnp.float32) + cnt_ref[...] * bf_ref[...]
        h = jnp.tanh(jnp.dot(a, wo1a_ref[...], preferred_element_type=jnp.float32)
                     + jnp.dot(own_ref[...], wo1b_ref[...],
                               preferred_element_type=jnp.float32)
                     + bo1_ref[...])
        new = jnp.dot(h, wo2_ref[...],
                      preferred_element_type=jnp.float32) + bo2_ref[...]
        if wnext is None:
            rest[-1][...] = new
        else:
            wn_ref, o_ref = rest
            o_ref[...] = jnp.dot(new, wn_ref[...],
                                 preferred_element_type=jnp.float32)

    args = list(aggs) + [cnt, own, wf, bf, wo1a, wo1b, bo1, wo2, bo2]
    specs = ([_tile_spec(tile, _EMB)] * nagg
             + [_tile_spec(tile, 1), _tile_spec(tile, _EMB)]
             + [_full_spec(a) for a in args[nagg + 2:]])
    if wnext is not None:
        args.append(wnext)
        specs.append(_full_spec(wnext))
    return pl.pallas_call(
        body,
        out_shape=jax.ShapeDtypeStruct((n, _EMB), jnp.float32),
        grid=(n // tile,),
        in_specs=specs,
        out_specs=_tile_spec(tile, _EMB),
        compiler_params=pltpu.CompilerParams(
            dimension_semantics=("parallel",)),
    )(*args)


def _pool_head_kernel(v_ref, starts_ref, ends_ref, recip_ref,
                      w1a_ref, b1a_ref, w1b_ref, b1b_ref,
                      w2ap_ref, w2ao_ref, b2a_ref,
                      w3ap_ref, w3ao_ref, b3a_ref,
                      w1bp_ref, w2bp_ref, w3bp_ref, bout_ref, o_ref):
    nvp = v_ref.shape[0]
    bsz = starts_ref.shape[0]
    r = jax.lax.broadcasted_iota(jnp.int32, (bsz, nvp), 1)
    inseg = (r >= starts_ref[...]) & (r < ends_ref[...])
    pool_w = jnp.where(inseg, recip_ref[...], 0.0)
    pred = jnp.dot(pool_w, v_ref[...], preferred_element_type=jnp.float32)
    tp = jnp.tanh(pred)
    h1 = jnp.tanh(jnp.dot(tp, w1a_ref[...],
                          preferred_element_type=jnp.float32) + b1a_ref[...])
    to1 = jnp.tanh(jnp.dot(h1, w1b_ref[...],
                           preferred_element_type=jnp.float32) + b1b_ref[...])
    h2 = jnp.tanh(jnp.dot(tp, w2ap_ref[...], preferred_element_type=jnp.float32)
                  + jnp.dot(to1, w2ao_ref[...], preferred_element_type=jnp.float32)
                  + b2a_ref[...])
    h3 = jnp.tanh(jnp.dot(tp, w3ap_ref[...], preferred_element_type=jnp.float32)
                  + jnp.dot(to1, w3ao_ref[...], preferred_element_type=jnp.float32)
                  + b3a_ref[...])
    o_ref[...] = (jnp.dot(h1, w1bp_ref[...], preferred_element_type=jnp.float32)
                  + jnp.dot(h2, w2bp_ref[...], preferred_element_type=jnp.float32)
                  + jnp.dot(h3, w3bp_ref[...], preferred_element_type=jnp.float32)
                  + bout_ref[...])


def _pool_and_head(v, starts_col, ends_col, recip_col, hp):
    bsz = starts_col.shape[0]
    args = (v, starts_col, ends_col, recip_col,
            hp['w1a'], hp['b1a'], hp['w1b'], hp['b1b'],
            hp['w2ap'], hp['w2ao'], hp['b2a'],
            hp['w3ap'], hp['w3ao'], hp['b3a'],
            hp['w1bp'], hp['w2bp'], hp['w3bp'], hp['bout'])
    vmem = pl.BlockSpec(memory_space=pltpu.MemorySpace.VMEM)
    return pl.pallas_call(
        _pool_head_kernel,
        out_shape=jax.ShapeDtypeStruct((bsz, _HEAD_W), jnp.float32),
        in_specs=[vmem] * len(args),
        out_specs=vmem,
    )(*args)


_EDGE_TILE = 512


_ACC_SPLIT = 2


def _edge_conv_body(masked, nacc):
    def body(*refs):
        if masked:
            tgt_ref, src_ref, ef_ref, val_ref, rt_ref, lt_ref, we_ref = refs[:7]
            acc_refs = refs[7:]
        else:
            tgt_ref, src_ref, ef_ref, rt_ref, lt_ref, we_ref = refs[:6]
            acc_refs = refs[6:]
        j = pl.program_id(1)

        @pl.when(j == 0)
        def _init():
            for a in acc_refs:
                a[...] = jnp.zeros(a.shape, jnp.float32)

        for mi in range(_EDGE_TILE):
            ti = tgt_ref[0, 0, mi]
            si = src_ref[0, 0, mi]
            e = ef_ref[0, 0, mi]
            val = jnp.tanh(rt_ref[ti] + lt_ref[si] + e * we_ref[...])
            if masked:
                val = val * val_ref[0, 0, mi]
            a = acc_refs[mi % nacc]
            a[ti] = a[ti] + val
    return body


def _edge_messages(rt, lt, we_row, tgt_idx, src_idx, ef, valid, nseg):
    nep = tgt_idx.shape[0]
    nblk = nep // _EDGE_TILE
    ncores = 2 if nblk % 2 == 0 else 1
    nblk2 = nblk // ncores
    tgt_b = tgt_idx.reshape(nblk, 1, _EDGE_TILE)
    src_b = src_idx.reshape(nblk, 1, _EDGE_TILE)
    ef_b = ef.reshape(nblk, 1, _EDGE_TILE)
    rt3 = rt.reshape(rt.shape[0], 1, _EMB)
    lt3 = lt.reshape(lt.shape[0], 1, _EMB)

    masked = valid is not None
    idx_spec = pl.BlockSpec((1, 1, _EDGE_TILE),
                            lambda c, j: (c * nblk2 + j, 0, 0),
                            memory_space=pltpu.MemorySpace.SMEM)
    args = [tgt_b, src_b, ef_b]
    specs = [idx_spec, idx_spec, idx_spec]
    if masked:
        args.append(valid.reshape(nblk, 1, _EDGE_TILE))
        specs.append(idx_spec)
    args += [rt3, lt3, we_row]
    specs += [pl.BlockSpec(rt3.shape, lambda c, j: (0, 0, 0)),
              pl.BlockSpec(lt3.shape, lambda c, j: (0, 0, 0)),
              pl.BlockSpec(we_row.shape, lambda c, j: (0, 0))]

    accs = pl.pallas_call(
        _edge_conv_body(masked, _ACC_SPLIT),
        out_shape=tuple(
            jax.ShapeDtypeStruct((ncores * nseg, 1, _EMB), jnp.float32)
            for _ in range(_ACC_SPLIT)),
        grid=(ncores, nblk2),
        in_specs=specs,
        out_specs=tuple(
            pl.BlockSpec((nseg, 1, _EMB), lambda c, j: (c, 0, 0))
            for _ in range(_ACC_SPLIT)),
        compiler_params=pltpu.CompilerParams(
            dimension_semantics=("parallel", "arbitrary")),
    )(*args)

    parts = []
    for acc in accs:
        a = acc.reshape(ncores, nseg, _EMB)
        for c in range(ncores):
            parts.append(a[c])
    return parts


def kernel(cons_feat, edge_indices, edge_feat, var_feat, n_cons_per_sample,
           n_vars_per_sample, ce_w1, ce_b1, ce_w2, ce_b2, ve_w1, ve_b1, ve_w2,
           ve_b2, cvc_ws, cvc_bs, cvc_wf, cvc_bf, cvc_wo1, cvc_bo1, cvc_wo2,
           cvc_bo2, ccv_ws, ccv_bs, ccv_wf, ccv_bf, ccv_wo1, ccv_bo1, ccv_wo2,
           ccv_bo2, hd_w1a, hd_b1a, hd_w1b, hd_b1b, hd_w2ap, hd_w2ao, hd_b2a,
           hd_w3ap, hd_w3ao, hd_b3a, hd_w1bp, hd_w2bp, hd_w3bp, hd_bout):
    del n_cons_per_sample
    nc, nv, ne = cons_feat.shape[0], var_feat.shape[0], edge_feat.shape[0]
    bsz = n_vars_per_sample.shape[0]

    ncp = _ceil_to(max(nc, 1), _ROW_TILE)
    nvp = _ceil_to(max(nv, 1), _ROW_TILE)
    nep = _ceil_to(max(ne, 1), _EDGE_TILE)

    c_in = jnp.pad(cons_feat.astype(jnp.float32), ((0, ncp - nc), (0, 0)))
    v_in = jnp.pad(var_feat.astype(jnp.float32), ((0, nvp - nv), (0, 0)))
    ef = jnp.pad(edge_feat.astype(jnp.float32), ((0, nep - ne), (0, 0)))
    cidx = jnp.pad(edge_indices[0].astype(jnp.int32), (0, nep - ne))
    vidx = jnp.pad(edge_indices[1].astype(jnp.int32), (0, nep - ne))
    if nep == ne:
        valid = None
        ones = jnp.ones((nep, 1), jnp.float32)
    else:
        valid = (jnp.arange(nep) < ne).astype(jnp.float32)[:, None]
        ones = valid

    wl1, we1, wr1 = cvc_ws[:_EMB], cvc_ws[_EMB:_EMB + 1], cvc_ws[_EMB + 1:]
    wl2, we2, wr2 = ccv_ws[:_EMB], ccv_ws[_EMB:_EMB + 1], ccv_ws[_EMB + 1:]

    c_emb, rt1 = _embed_and_project(c_in, ce_w1, ce_b1, ce_w2, ce_b2,
                                    [(wl1, cvc_bs)])
    v_emb, lt1, rt2 = _embed_and_project(v_in, ve_w1, ve_b1, ve_w2, ve_b2,
                                         [(wr1, None), (wl2, ccv_bs)])

    cnt_c = jax.ops.segment_sum(ones, cidx, num_segments=ncp)
    cnt_v = jax.ops.segment_sum(ones, vidx, num_segments=nvp)

    aggs1 = _edge_messages(rt1, lt1, we1, cidx, vidx, ef, valid, ncp)
    lt2 = _conv_out_call(aggs1, cnt_c, c_emb, cvc_wf, cvc_bf,
                         cvc_wo1[:_EMB], cvc_wo1[_EMB:], cvc_bo1,
                         cvc_wo2, cvc_bo2, wnext=wr2)

    aggs2 = _edge_messages(rt2, lt2, we2, vidx, cidx, ef, valid, nvp)
    v2 = _conv_out_call(aggs2, cnt_v, v_emb, ccv_wf, ccv_bf,
                        ccv_wo1[:_EMB], ccv_wo1[_EMB:], ccv_bo1,
                        ccv_wo2, ccv_bo2, wnext=None)

    nvars = n_vars_per_sample.astype(jnp.int32)
    ends = jnp.cumsum(nvars)
    starts_col = (ends - nvars).reshape(bsz, 1)
    ends_col = ends.reshape(bsz, 1)
    recip_col = (1.0 / jnp.maximum(nvars, 1).astype(jnp.float32)).reshape(bsz, 1)
    hp = dict(w1a=hd_w1a, b1a=hd_b1a, w1b=hd_w1b, b1b=hd_b1b,
              w2ap=hd_w2ap, w2ao=hd_w2ao, b2a=hd_b2a,
              w3ap=hd_w3ap, w3ao=hd_w3ao, b3a=hd_b3a,
              w1bp=hd_w1bp, w2bp=hd_w2bp, w3bp=hd_w3bp, bout=hd_bout)
    out = _pool_and_head(v2, starts_col, ends_col, recip_col, hp)
    return out[:, :_OUT_COLS]


# speedup vs baseline: 1.2304x; 1.2304x over previous
import jax
import jax.numpy as jnp
from jax.experimental import pallas as pl
from jax.experimental.pallas import tpu as pltpu

_EMB = 64
_ROW_TILE = 512
_HEAD_W = 128
_OUT_COLS = 14 + 56 + 56


def _ceil_to(n, m):
    return ((n + m - 1) // m) * m


def _tile_spec(tile, cols):
    return pl.BlockSpec((tile, cols), lambda i: (i, 0))


def _full_spec(arr):
    return pl.BlockSpec(arr.shape, lambda i: (0,) * arr.ndim)


def _embed_and_project(x, w1, b1, w2, b2, projs, tile=_ROW_TILE):
    n = x.shape[0]
    nproj = len(projs)
    has_bias = [pb is not None for (_, pb) in projs]
    args = [x, w1, b1, w2, b2]
    specs = [_tile_spec(tile, x.shape[1]), _full_spec(w1), _full_spec(b1),
             _full_spec(w2), _full_spec(b2)]
    for (pw, pb) in projs:
        args.append(pw)
        specs.append(_full_spec(pw))
        if pb is not None:
            args.append(pb)
            specs.append(_full_spec(pb))

    def body(x_ref, w1_ref, b1_ref, w2_ref, b2_ref, *rest):
        nin = sum(1 + int(hb) for hb in has_bias)
        in_it = iter(rest[:nin])
        outs = rest[nin:]
        h = jnp.tanh(jnp.dot(x_ref[...], w1_ref[...],
                             preferred_element_type=jnp.float32) + b1_ref[...])
        emb = jnp.tanh(jnp.dot(h, w2_ref[...],
                               preferred_element_type=jnp.float32) + b2_ref[...])
        outs[0][...] = emb
        for k in range(nproj):
            wref = next(in_it)
            p = jnp.dot(emb, wref[...], preferred_element_type=jnp.float32)
            if has_bias[k]:
                p = p + next(in_it)[...]
            outs[1 + k][...] = p

    out_shapes = tuple(jax.ShapeDtypeStruct((n, _EMB), jnp.float32)
                       for _ in range(1 + nproj))
    out_specs = tuple(_tile_spec(tile, _EMB) for _ in range(1 + nproj))
    return pl.pallas_call(
        body,
        out_shape=out_shapes,
        grid=(n // tile,),
        in_specs=specs,
        out_specs=out_specs,
        compiler_params=pltpu.CompilerParams(
            dimension_semantics=("parallel",)),
    )(*args)


def _conv_out_call(aggs, cnt, own, wf, bf, wo1a, wo1b, bo1, wo2, bo2,
                   wnext=None, tile=_ROW_TILE):
    nagg = len(aggs)
    n = aggs[0].shape[0]

    def body(*refs):
        agg_refs = refs[:nagg]
        (cnt_ref, own_ref, wf_ref, bf_ref, wo1a_ref, wo1b_ref,
         bo1_ref, wo2_ref, bo2_ref) = refs[nagg:nagg + 9]
        rest = refs[nagg + 9:]
        araw = agg_refs[0][...]
        for k in range(1, nagg):
            araw = araw + agg_refs[k][...]
        a = jnp.dot(araw, wf_ref[...],
                    preferred_element_type=jnp.float32) + cnt_ref[...] * bf_ref[...]
        h = jnp.tanh(jnp.dot(a, wo1a_ref[...], preferred_element_type=jnp.float32)
                     + jnp.dot(own_ref[...], wo1b_ref[...],
                               preferred_element_type=jnp.float32)
                     + bo1_ref[...])
        new = jnp.dot(h, wo2_ref[...],
                      preferred_element_type=jnp.float32) + bo2_ref[...]
        if wnext is None:
            rest[-1][...] = new
        else:
            wn_ref, o_ref = rest
            o_ref[...] = jnp.dot(new, wn_ref[...],
                                 preferred_element_type=jnp.float32)

    args = list(aggs) + [cnt, own, wf, bf, wo1a, wo1b, bo1, wo2, bo2]
    specs = ([_tile_spec(tile, _EMB)] * nagg
             + [_tile_spec(tile, 1), _tile_spec(tile, _EMB)]
             + [_full_spec(a) for a in args[nagg + 2:]])
    if wnext is not None:
        args.append(wnext)
        specs.append(_full_spec(wnext))
    return pl.pallas_call(
        body,
        out_shape=jax.ShapeDtypeStruct((n, _EMB), jnp.float32),
        grid=(n // tile,),
        in_specs=specs,
        out_specs=_tile_spec(tile, _EMB),
        compiler_params=pltpu.CompilerParams(
            dimension_semantics=("parallel",)),
    )(*args)


def _pool_head_kernel(v_ref, starts_ref, ends_ref, recip_ref,
                      w1a_ref, b1a_ref, w1b_ref, b1b_ref,
                      w2ap_ref, w2ao_ref, b2a_ref,
                      w3ap_ref, w3ao_ref, b3a_ref,
                      w1bp_ref, w2bp_ref, w3bp_ref, bout_ref, o_ref):
    nvp = v_ref.shape[0]
    bsz = starts_ref.shape[0]
    r = jax.lax.broadcasted_iota(jnp.int32, (bsz, nvp), 1)
    inseg = (r >= starts_ref[...]) & (r < ends_ref[...])
    pool_w = jnp.where(inseg, recip_ref[...], 0.0)
    pred = jnp.dot(pool_w, v_ref[...], preferred_element_type=jnp.float32)
    tp = jnp.tanh(pred)
    h1 = jnp.tanh(jnp.dot(tp, w1a_ref[...],
                          preferred_element_type=jnp.float32) + b1a_ref[...])
    to1 = jnp.tanh(jnp.dot(h1, w1b_ref[...],
                           preferred_element_type=jnp.float32) + b1b_ref[...])
    h2 = jnp.tanh(jnp.dot(tp, w2ap_ref[...], preferred_element_type=jnp.float32)
                  + jnp.dot(to1, w2ao_ref[...], preferred_element_type=jnp.float32)
                  + b2a_ref[...])
    h3 = jnp.tanh(jnp.dot(tp, w3ap_ref[...], preferred_element_type=jnp.float32)
                  + jnp.dot(to1, w3ao_ref[...], preferred_element_type=jnp.float32)
                  + b3a_ref[...])
    o_ref[...] = (jnp.dot(h1, w1bp_ref[...], preferred_element_type=jnp.float32)
                  + jnp.dot(h2, w2bp_ref[...], preferred_element_type=jnp.float32)
                  + jnp.dot(h3, w3bp_ref[...], preferred_element_type=jnp.float32)
                  + bout_ref[...])


def _pool_and_head(v, starts_col, ends_col, recip_col, hp):
    bsz = starts_col.shape[0]
    args = (v, starts_col, ends_col, recip_col,
            hp['w1a'], hp['b1a'], hp['w1b'], hp['b1b'],
            hp['w2ap'], hp['w2ao'], hp['b2a'],
            hp['w3ap'], hp['w3ao'], hp['b3a'],
            hp['w1bp'], hp['w2bp'], hp['w3bp'], hp['bout'])
    vmem = pl.BlockSpec(memory_space=pltpu.MemorySpace.VMEM)
    return pl.pallas_call(
        _pool_head_kernel,
        out_shape=jax.ShapeDtypeStruct((bsz, _HEAD_W), jnp.float32),
        in_specs=[vmem] * len(args),
        out_specs=vmem,
    )(*args)


_EDGE_TILE = 512


_ACC_SPLIT = 4
_WRITE_STEPS = 8


def _edge_conv_body(masked, nacc, nblk2, rows_blk):
    def body(*refs):
        if masked:
            tgt_ref, src_ref, ef_ref, val_ref, rt_ref, lt_ref, we_ref = refs[:7]
            rest = refs[7:]
        else:
            tgt_ref, src_ref, ef_ref, rt_ref, lt_ref, we_ref = refs[:6]
            rest = refs[6:]
        o_ref = rest[0]
        acc_refs = rest[1:]
        j = pl.program_id(1)

        @pl.when(j == 0)
        def _init():
            for a in acc_refs:
                a[...] = jnp.zeros(a.shape, jnp.float32)

        @pl.when(j < nblk2)
        def _accumulate():
            for mi in range(_EDGE_TILE):
                ti = tgt_ref[0, 0, mi]
                si = src_ref[0, 0, mi]
                e = ef_ref[0, 0, mi]
                val = jnp.tanh(rt_ref[ti] + lt_ref[si] + e * we_ref[...])
                if masked:
                    val = val * val_ref[0, 0, mi]
                a = acc_refs[mi % nacc]
                a[ti] = a[ti] + val

        @pl.when(j >= nblk2)
        def _writeout():
            w = j - nblk2
            sl = pl.ds(w * rows_blk, rows_blk)
            tot = acc_refs[0][sl]
            for a in acc_refs[1:]:
                tot = tot + a[sl]
            o_ref[...] = tot
    return body


def _edge_messages(rt, lt, we_row, tgt_idx, src_idx, ef, valid, nseg):
    nep = tgt_idx.shape[0]
    nblk = nep // _EDGE_TILE
    ncores = 2 if nblk % 2 == 0 else 1
    nblk2 = nblk // ncores
    nw = _WRITE_STEPS if nseg % _WRITE_STEPS == 0 else 1
    rows_blk = nseg // nw
    tgt_b = tgt_idx.reshape(nblk, 1, _EDGE_TILE)
    src_b = src_idx.reshape(nblk, 1, _EDGE_TILE)
    ef_b = ef.reshape(nblk, 1, _EDGE_TILE)
    rt3 = rt.reshape(rt.shape[0], 1, _EMB)
    lt3 = lt.reshape(lt.shape[0], 1, _EMB)

    masked = valid is not None

    def idx_map(c, j):
        return (c * nblk2 + jnp.minimum(j, nblk2 - 1), 0, 0)

    idx_spec = pl.BlockSpec((1, 1, _EDGE_TILE), idx_map,
                            memory_space=pltpu.MemorySpace.SMEM)
    args = [tgt_b, src_b, ef_b]
    specs = [idx_spec, idx_spec, idx_spec]
    if masked:
        args.append(valid.reshape(nblk, 1, _EDGE_TILE))
        specs.append(idx_spec)
    args += [rt3, lt3, we_row]
    specs += [pl.BlockSpec(rt3.shape, lambda c, j: (0, 0, 0)),
              pl.BlockSpec(lt3.shape, lambda c, j: (0, 0, 0)),
              pl.BlockSpec(we_row.shape, lambda c, j: (0, 0))]

    def out_map(c, j):
        return (c * nw + jnp.clip(j - nblk2, 0, nw - 1), 0, 0)

    acc = pl.pallas_call(
        _edge_conv_body(masked, _ACC_SPLIT, nblk2, rows_blk),
        out_shape=jax.ShapeDtypeStruct((ncores * nseg, 1, _EMB),
                                       jnp.float32),
        grid=(ncores, nblk2 + nw),
        in_specs=specs,
        out_specs=pl.BlockSpec((rows_blk, 1, _EMB), out_map),
        scratch_shapes=[pltpu.VMEM((nseg, 1, _EMB), jnp.float32)
                        for _ in range(_ACC_SPLIT)],
        compiler_params=pltpu.CompilerParams(
            dimension_semantics=("parallel", "arbitrary")),
    )(*args)

    a = acc.reshape(ncores, nseg, _EMB)
    return [a[c] for c in range(ncores)]


def kernel(cons_feat, edge_indices, edge_feat, var_feat, n_cons_per_sample,
           n_vars_per_sample, ce_w1, ce_b1, ce_w2, ce_b2, ve_w1, ve_b1, ve_w2,
           ve_b2, cvc_ws, cvc_bs, cvc_wf, cvc_bf, cvc_wo1, cvc_bo1, cvc_wo2,
           cvc_bo2, ccv_ws, ccv_bs, ccv_wf, ccv_bf, ccv_wo1, ccv_bo1, ccv_wo2,
           ccv_bo2, hd_w1a, hd_b1a, hd_w1b, hd_b1b, hd_w2ap, hd_w2ao, hd_b2a,
           hd_w3ap, hd_w3ao, hd_b3a, hd_w1bp, hd_w2bp, hd_w3bp, hd_bout):
    del n_cons_per_sample
    nc, nv, ne = cons_feat.shape[0], var_feat.shape[0], edge_feat.shape[0]
    bsz = n_vars_per_sample.shape[0]

    ncp = _ceil_to(max(nc, 1), _ROW_TILE)
    nvp = _ceil_to(max(nv, 1), _ROW_TILE)
    nep = _ceil_to(max(ne, 1), _EDGE_TILE)

    c_in = jnp.pad(cons_feat.astype(jnp.float32), ((0, ncp - nc), (0, 0)))
    v_in = jnp.pad(var_feat.astype(jnp.float32), ((0, nvp - nv), (0, 0)))
    ef = jnp.pad(edge_feat.astype(jnp.float32), ((0, nep - ne), (0, 0)))
    cidx = jnp.pad(edge_indices[0].astype(jnp.int32), (0, nep - ne))
    vidx = jnp.pad(edge_indices[1].astype(jnp.int32), (0, nep - ne))
    if nep == ne:
        valid = None
        ones = jnp.ones((nep, 1), jnp.float32)
    else:
        valid = (jnp.arange(nep) < ne).astype(jnp.float32)[:, None]
        ones = valid

    wl1, we1, wr1 = cvc_ws[:_EMB], cvc_ws[_EMB:_EMB + 1], cvc_ws[_EMB + 1:]
    wl2, we2, wr2 = ccv_ws[:_EMB], ccv_ws[_EMB:_EMB + 1], ccv_ws[_EMB + 1:]

    c_emb, rt1 = _embed_and_project(c_in, ce_w1, ce_b1, ce_w2, ce_b2,
                                    [(wl1, cvc_bs)])
    v_emb, lt1, rt2 = _embed_and_project(v_in, ve_w1, ve_b1, ve_w2, ve_b2,
                                         [(wr1, None), (wl2, ccv_bs)])

    cnt_c = jax.ops.segment_sum(ones, cidx, num_segments=ncp)
    cnt_v = jax.ops.segment_sum(ones, vidx, num_segments=nvp)

    aggs1 = _edge_messages(rt1, lt1, we1, cidx, vidx, ef, valid, ncp)
    lt2 = _conv_out_call(aggs1, cnt_c, c_emb, cvc_wf, cvc_bf,
                         cvc_wo1[:_EMB], cvc_wo1[_EMB:], cvc_bo1,
                         cvc_wo2, cvc_bo2, wnext=wr2)

    aggs2 = _edge_messages(rt2, lt2, we2, vidx, cidx, ef, valid, nvp)
    v2 = _conv_out_call(aggs2, cnt_v, v_emb, ccv_wf, ccv_bf,
                        ccv_wo1[:_EMB], ccv_wo1[_EMB:], ccv_bo1,
                        ccv_wo2, ccv_bo2, wnext=None)

    nvars = n_vars_per_sample.astype(jnp.int32)
    ends = jnp.cumsum(nvars)
    starts_col = (ends - nvars).reshape(bsz, 1)
    ends_col = ends.reshape(bsz, 1)
    recip_col = (1.0 / jnp.maximum(nvars, 1).astype(jnp.float32)).reshape(bsz, 1)
    hp = dict(w1a=hd_w1a, b1a=hd_b1a, w1b=hd_w1b, b1b=hd_b1b,
              w2ap=hd_w2ap, w2ao=hd_w2ao, b2a=hd_b2a,
              w3ap=hd_w3ap, w3ao=hd_w3ao, b3a=hd_b3a,
              w1bp=hd_w1bp, w2bp=hd_w2bp, w3bp=hd_w3bp, bout=hd_bout)
    out = _pool_and_head(v2, starts_col, ends_col, recip_col, hp)
    return out[:, :_OUT_COLS]


# speedup vs baseline: 1.2346x; 1.0034x over previous
import jax
import jax.numpy as jnp
from jax.experimental import pallas as pl
from jax.experimental.pallas import tpu as pltpu

_EMB = 64
_ROW_TILE = 512
_HEAD_W = 128
_OUT_COLS = 14 + 56 + 56


def _ceil_to(n, m):
    return ((n + m - 1) // m) * m


def _tile_spec(tile, cols):
    return pl.BlockSpec((tile, cols), lambda i: (i, 0))


def _full_spec(arr):
    return pl.BlockSpec(arr.shape, lambda i: (0,) * arr.ndim)


def _embed_and_project(x, w1, b1, w2, b2, projs, tile=_ROW_TILE):
    n = x.shape[0]
    nproj = len(projs)
    has_bias = [pb is not None for (_, pb) in projs]
    args = [x, w1, b1, w2, b2]
    specs = [_tile_spec(tile, x.shape[1]), _full_spec(w1), _full_spec(b1),
             _full_spec(w2), _full_spec(b2)]
    for (pw, pb) in projs:
        args.append(pw)
        specs.append(_full_spec(pw))
        if pb is not None:
            args.append(pb)
            specs.append(_full_spec(pb))

    def body(x_ref, w1_ref, b1_ref, w2_ref, b2_ref, *rest):
        nin = sum(1 + int(hb) for hb in has_bias)
        in_it = iter(rest[:nin])
        outs = rest[nin:]
        h = jnp.tanh(jnp.dot(x_ref[...], w1_ref[...],
                             preferred_element_type=jnp.float32) + b1_ref[...])
        emb = jnp.tanh(jnp.dot(h, w2_ref[...],
                               preferred_element_type=jnp.float32) + b2_ref[...])
        outs[0][...] = emb
        for k in range(nproj):
            wref = next(in_it)
            p = jnp.dot(emb, wref[...], preferred_element_type=jnp.float32)
            if has_bias[k]:
                p = p + next(in_it)[...]
            outs[1 + k][...] = p

    out_shapes = tuple(jax.ShapeDtypeStruct((n, _EMB), jnp.float32)
                       for _ in range(1 + nproj))
    out_specs = tuple(_tile_spec(tile, _EMB) for _ in range(1 + nproj))
    return pl.pallas_call(
        body,
        out_shape=out_shapes,
        grid=(n // tile,),
        in_specs=specs,
        out_specs=out_specs,
        compiler_params=pltpu.CompilerParams(
            dimension_semantics=("parallel",)),
    )(*args)


def _conv_out_call(aggs, cnt, own, wf, bf, wo1a, wo1b, bo1, wo2, bo2,
                   wnext=None, tile=_ROW_TILE):
    nagg = len(aggs)
    n = aggs[0].shape[0]

    def body(*refs):
        agg_refs = refs[:nagg]
        (cnt_ref, own_ref, wf_ref, bf_ref, wo1a_ref, wo1b_ref,
         bo1_ref, wo2_ref, bo2_ref) = refs[nagg:nagg + 9]
        rest = refs[nagg + 9:]
        araw = agg_refs[0][...]
        for k in range(1, nagg):
            araw = araw + agg_refs[k][...]
        a = jnp.dot(araw, wf_ref[...],
                    preferred_element_type=jnp.float32) + cnt_ref[...] * bf_ref[...]
        h = jnp.tanh(jnp.dot(a, wo1a_ref[...], preferred_element_type=jnp.float32)
                     + jnp.dot(own_ref[...], wo1b_ref[...],
                               preferred_element_type=jnp.float32)
                     + bo1_ref[...])
        new = jnp.dot(h, wo2_ref[...],
                      preferred_element_type=jnp.float32) + bo2_ref[...]
        if wnext is None:
            rest[-1][...] = new
        else:
            wn_ref, o_ref = rest
            o_ref[...] = jnp.dot(new, wn_ref[...],
                                 preferred_element_type=jnp.float32)

    args = list(aggs) + [cnt, own, wf, bf, wo1a, wo1b, bo1, wo2, bo2]
    specs = ([_tile_spec(tile, _EMB)] * nagg
             + [_tile_spec(tile, 1), _tile_spec(tile, _EMB)]
             + [_full_spec(a) for a in args[nagg + 2:]])
    if wnext is not None:
        args.append(wnext)
        specs.append(_full_spec(wnext))
    return pl.pallas_call(
        body,
        out_shape=jax.ShapeDtypeStruct((n, _EMB), jnp.float32),
        grid=(n // tile,),
        in_specs=specs,
        out_specs=_tile_spec(tile, _EMB),
        compiler_params=pltpu.CompilerParams(
            dimension_semantics=("parallel",)),
    )(*args)


def _pool_head_kernel(v_ref, starts_ref, ends_ref, recip_ref,
                      w1a_ref, b1a_ref, w1b_ref, b1b_ref,
                      w2ap_ref, w2ao_ref, b2a_ref,
                      w3ap_ref, w3ao_ref, b3a_ref,
                      w1bp_ref, w2bp_ref, w3bp_ref, bout_ref, o_ref):
    nvp = v_ref.shape[0]
    bsz = starts_ref.shape[0]
    r = jax.lax.broadcasted_iota(jnp.int32, (bsz, nvp), 1)
    inseg = (r >= starts_ref[...]) & (r < ends_ref[...])
    pool_w = jnp.where(inseg, recip_ref[...], 0.0)
    pred = jnp.dot(pool_w, v_ref[...], preferred_element_type=jnp.float32)
    tp = jnp.tanh(pred)
    h1 = jnp.tanh(jnp.dot(tp, w1a_ref[...],
                          preferred_element_type=jnp.float32) + b1a_ref[...])
    to1 = jnp.tanh(jnp.dot(h1, w1b_ref[...],
                           preferred_element_type=jnp.float32) + b1b_ref[...])
    h2 = jnp.tanh(jnp.dot(tp, w2ap_ref[...], preferred_element_type=jnp.float32)
                  + jnp.dot(to1, w2ao_ref[...], preferred_element_type=jnp.float32)
                  + b2a_ref[...])
    h3 = jnp.tanh(jnp.dot(tp, w3ap_ref[...], preferred_element_type=jnp.float32)
                  + jnp.dot(to1, w3ao_ref[...], preferred_element_type=jnp.float32)
                  + b3a_ref[...])
    o_ref[...] = (jnp.dot(h1, w1bp_ref[...], preferred_element_type=jnp.float32)
                  + jnp.dot(h2, w2bp_ref[...], preferred_element_type=jnp.float32)
                  + jnp.dot(h3, w3bp_ref[...], preferred_element_type=jnp.float32)
                  + bout_ref[...])


def _pool_and_head(v, starts_col, ends_col, recip_col, hp):
    bsz = starts_col.shape[0]
    args = (v, starts_col, ends_col, recip_col,
            hp['w1a'], hp['b1a'], hp['w1b'], hp['b1b'],
            hp['w2ap'], hp['w2ao'], hp['b2a'],
            hp['w3ap'], hp['w3ao'], hp['b3a'],
            hp['w1bp'], hp['w2bp'], hp['w3bp'], hp['bout'])
    vmem = pl.BlockSpec(memory_space=pltpu.MemorySpace.VMEM)
    return pl.pallas_call(
        _pool_head_kernel,
        out_shape=jax.ShapeDtypeStruct((bsz, _HEAD_W), jnp.float32),
        in_specs=[vmem] * len(args),
        out_specs=vmem,
    )(*args)


_EDGE_TILE = 1024


_ACC_SPLIT = 5
_WRITE_STEPS = 8


def _edge_conv_body(masked, nacc, nblk2, rows_blk):
    def body(*refs):
        if masked:
            tgt_ref, src_ref, ef_ref, val_ref, rt_ref, lt_ref, we_ref = refs[:7]
            rest = refs[7:]
        else:
            tgt_ref, src_ref, ef_ref, rt_ref, lt_ref, we_ref = refs[:6]
            rest = refs[6:]
        o_ref = rest[0]
        acc_refs = rest[1:]
        j = pl.program_id(1)

        @pl.when(j == 0)
        def _init():
            for a in acc_refs:
                a[...] = jnp.zeros(a.shape, jnp.float32)

        @pl.when(j < nblk2)
        def _accumulate():
            for mi in range(_EDGE_TILE):
                ti = tgt_ref[0, 0, mi]
                si = src_ref[0, 0, mi]
                e = ef_ref[0, 0, mi]
                val = jnp.tanh(rt_ref[ti] + lt_ref[si] + e * we_ref[...])
                if masked:
                    val = val * val_ref[0, 0, mi]
                a = acc_refs[mi % nacc]
                a[ti] = a[ti] + val

        @pl.when(j >= nblk2)
        def _writeout():
            w = j - nblk2
            sl = pl.ds(w * rows_blk, rows_blk)
            tot = acc_refs[0][sl]
            for a in acc_refs[1:]:
                tot = tot + a[sl]
            o_ref[...] = tot
    return body


def _edge_messages(rt, lt, we_row, tgt_idx, src_idx, ef, valid, nseg):
    nep = tgt_idx.shape[0]
    nblk = nep // _EDGE_TILE
    ncores = 2 if nblk % 2 == 0 else 1
    nblk2 = nblk // ncores
    nw = _WRITE_STEPS if nseg % _WRITE_STEPS == 0 else 1
    rows_blk = nseg // nw
    tgt_b = tgt_idx.reshape(nblk, 1, _EDGE_TILE)
    src_b = src_idx.reshape(nblk, 1, _EDGE_TILE)
    ef_b = ef.reshape(nblk, 1, _EDGE_TILE)
    rt3 = rt.reshape(rt.shape[0], 1, _EMB)
    lt3 = lt.reshape(lt.shape[0], 1, _EMB)

    masked = valid is not None

    def idx_map(c, j):
        return (c * nblk2 + jnp.minimum(j, nblk2 - 1), 0, 0)

    idx_spec = pl.BlockSpec((1, 1, _EDGE_TILE), idx_map,
                            memory_space=pltpu.MemorySpace.SMEM)
    args = [tgt_b, src_b, ef_b]
    specs = [idx_spec, idx_spec, idx_spec]
    if masked:
        args.append(valid.reshape(nblk, 1, _EDGE_TILE))
        specs.append(idx_spec)
    args += [rt3, lt3, we_row]
    specs += [pl.BlockSpec(rt3.shape, lambda c, j: (0, 0, 0)),
              pl.BlockSpec(lt3.shape, lambda c, j: (0, 0, 0)),
              pl.BlockSpec(we_row.shape, lambda c, j: (0, 0))]

    def out_map(c, j):
        return (c * nw + jnp.clip(j - nblk2, 0, nw - 1), 0, 0)

    acc = pl.pallas_call(
        _edge_conv_body(masked, _ACC_SPLIT, nblk2, rows_blk),
        out_shape=jax.ShapeDtypeStruct((ncores * nseg, 1, _EMB),
                                       jnp.float32),
        grid=(ncores, nblk2 + nw),
        in_specs=specs,
        out_specs=pl.BlockSpec((rows_blk, 1, _EMB), out_map),
        scratch_shapes=[pltpu.VMEM((nseg, 1, _EMB), jnp.float32)
                        for _ in range(_ACC_SPLIT)],
        compiler_params=pltpu.CompilerParams(
            dimension_semantics=("parallel", "arbitrary")),
    )(*args)

    a = acc.reshape(ncores, nseg, _EMB)
    return [a[c] for c in range(ncores)]


def kernel(cons_feat, edge_indices, edge_feat, var_feat, n_cons_per_sample,
           n_vars_per_sample, ce_w1, ce_b1, ce_w2, ce_b2, ve_w1, ve_b1, ve_w2,
           ve_b2, cvc_ws, cvc_bs, cvc_wf, cvc_bf, cvc_wo1, cvc_bo1, cvc_wo2,
           cvc_bo2, ccv_ws, ccv_bs, ccv_wf, ccv_bf, ccv_wo1, ccv_bo1, ccv_wo2,
           ccv_bo2, hd_w1a, hd_b1a, hd_w1b, hd_b1b, hd_w2ap, hd_w2ao, hd_b2a,
           hd_w3ap, hd_w3ao, hd_b3a, hd_w1bp, hd_w2bp, hd_w3bp, hd_bout):
    del n_cons_per_sample
    nc, nv, ne = cons_feat.shape[0], var_feat.shape[0], edge_feat.shape[0]
    bsz = n_vars_per_sample.shape[0]

    ncp = _ceil_to(max(nc, 1), _ROW_TILE)
    nvp = _ceil_to(max(nv, 1), _ROW_TILE)
    nep = _ceil_to(max(ne, 1), _EDGE_TILE)

    c_in = jnp.pad(cons_feat.astype(jnp.float32), ((0, ncp - nc), (0, 0)))
    v_in = jnp.pad(var_feat.astype(jnp.float32), ((0, nvp - nv), (0, 0)))
    ef = jnp.pad(edge_feat.astype(jnp.float32), ((0, nep - ne), (0, 0)))
    cidx = jnp.pad(edge_indices[0].astype(jnp.int32), (0, nep - ne))
    vidx = jnp.pad(edge_indices[1].astype(jnp.int32), (0, nep - ne))
    if nep == ne:
        valid = None
        ones = jnp.ones((nep, 1), jnp.float32)
    else:
        valid = (jnp.arange(nep) < ne).astype(jnp.float32)[:, None]
        ones = valid

    wl1, we1, wr1 = cvc_ws[:_EMB], cvc_ws[_EMB:_EMB + 1], cvc_ws[_EMB + 1:]
    wl2, we2, wr2 = ccv_ws[:_EMB], ccv_ws[_EMB:_EMB + 1], ccv_ws[_EMB + 1:]

    c_emb, rt1 = _embed_and_project(c_in, ce_w1, ce_b1, ce_w2, ce_b2,
                                    [(wl1, cvc_bs)])
    v_emb, lt1, rt2 = _embed_and_project(v_in, ve_w1, ve_b1, ve_w2, ve_b2,
                                         [(wr1, None), (wl2, ccv_bs)])

    cnt_c = jax.ops.segment_sum(ones, cidx, num_segments=ncp)
    cnt_v = jax.ops.segment_sum(ones, vidx, num_segments=nvp)

    aggs1 = _edge_messages(rt1, lt1, we1, cidx, vidx, ef, valid, ncp)
    lt2 = _conv_out_call(aggs1, cnt_c, c_emb, cvc_wf, cvc_bf,
                         cvc_wo1[:_EMB], cvc_wo1[_EMB:], cvc_bo1,
                         cvc_wo2, cvc_bo2, wnext=wr2)

    aggs2 = _edge_messages(rt2, lt2, we2, vidx, cidx, ef, valid, nvp)
    v2 = _conv_out_call(aggs2, cnt_v, v_emb, ccv_wf, ccv_bf,
                        ccv_wo1[:_EMB], ccv_wo1[_EMB:], ccv_bo1,
                        ccv_wo2, ccv_bo2, wnext=None)

    nvars = n_vars_per_sample.astype(jnp.int32)
    ends = jnp.cumsum(nvars)
    starts_col = (ends - nvars).reshape(bsz, 1)
    ends_col = ends.reshape(bsz, 1)
    recip_col = (1.0 / jnp.maximum(nvars, 1).astype(jnp.float32)).reshape(bsz, 1)
    hp = dict(w1a=hd_w1a, b1a=hd_b1a, w1b=hd_w1b, b1b=hd_b1b,
              w2ap=hd_w2ap, w2ao=hd_w2ao, b2a=hd_b2a,
              w3ap=hd_w3ap, w3ao=hd_w3ao, b3a=hd_b3a,
              w1bp=hd_w1bp, w2bp=hd_w2bp, w3bp=hd_w3bp, bout=hd_bout)
    out = _pool_and_head(v2, starts_col, ends_col, recip_col, hp)
    return out[:, :_OUT_COLS]


# speedup vs baseline: 1.2347x; 1.0001x over previous
import jax
import jax.numpy as jnp
from jax.experimental import pallas as pl
from jax.experimental.pallas import tpu as pltpu

_EMB = 64
_ROW_TILE = 512
_HEAD_W = 128
_OUT_COLS = 14 + 56 + 56


def _ceil_to(n, m):
    return ((n + m - 1) // m) * m


def _tile_spec(tile, cols):
    return pl.BlockSpec((tile, cols), lambda i: (i, 0))


def _full_spec(arr):
    return pl.BlockSpec(arr.shape, lambda i: (0,) * arr.ndim)


def _embed_and_project(x, w1, b1, w2, b2, projs, tile=_ROW_TILE):
    n = x.shape[0]
    nproj = len(projs)
    has_bias = [pb is not None for (_, pb) in projs]
    args = [x, w1, b1, w2, b2]
    specs = [_tile_spec(tile, x.shape[1]), _full_spec(w1), _full_spec(b1),
             _full_spec(w2), _full_spec(b2)]
    for (pw, pb) in projs:
        args.append(pw)
        specs.append(_full_spec(pw))
        if pb is not None:
            args.append(pb)
            specs.append(_full_spec(pb))

    def body(x_ref, w1_ref, b1_ref, w2_ref, b2_ref, *rest):
        nin = sum(1 + int(hb) for hb in has_bias)
        in_it = iter(rest[:nin])
        outs = rest[nin:]
        h = jnp.tanh(jnp.dot(x_ref[...], w1_ref[...],
                             preferred_element_type=jnp.float32) + b1_ref[...])
        emb = jnp.tanh(jnp.dot(h, w2_ref[...],
                               preferred_element_type=jnp.float32) + b2_ref[...])
        outs[0][...] = emb
        for k in range(nproj):
            wref = next(in_it)
            p = jnp.dot(emb, wref[...], preferred_element_type=jnp.float32)
            if has_bias[k]:
                p = p + next(in_it)[...]
            outs[1 + k][...] = p

    out_shapes = tuple(jax.ShapeDtypeStruct((n, _EMB), jnp.float32)
                       for _ in range(1 + nproj))
    out_specs = tuple(_tile_spec(tile, _EMB) for _ in range(1 + nproj))
    return pl.pallas_call(
        body,
        out_shape=out_shapes,
        grid=(n // tile,),
        in_specs=specs,
        out_specs=out_specs,
        compiler_params=pltpu.CompilerParams(
            dimension_semantics=("parallel",)),
    )(*args)


def _conv_out_call(aggs, cnt, own, wf, bf, wo1a, wo1b, bo1, wo2, bo2,
                   wnext=None, tile=_ROW_TILE):
    nagg = len(aggs)
    n = aggs[0].shape[0]

    def body(*refs):
        agg_refs = refs[:nagg]
        (cnt_ref, own_ref, wf_ref, bf_ref, wo1a_ref, wo1b_ref,
         bo1_ref, wo2_ref, bo2_ref) = refs[nagg:nagg + 9]
        rest = refs[nagg + 9:]
        araw = agg_refs[0][...]
        for k in range(1, nagg):
            araw = araw + agg_refs[k][...]
        a = jnp.dot(araw, wf_ref[...],
                    preferred_element_type=jnp.float32) + cnt_ref[...] * bf_ref[...]
        h = jnp.tanh(jnp.dot(a, wo1a_ref[...], preferred_element_type=jnp.float32)
                     + jnp.dot(own_ref[...], wo1b_ref[...],
                               preferred_element_type=jnp.float32)
                     + bo1_ref[...])
        new = jnp.dot(h, wo2_ref[...],
                      preferred_element_type=jnp.float32) + bo2_ref[...]
        if wnext is None:
            rest[-1][...] = new
        else:
            wn_ref, o_ref = rest
            o_ref[...] = jnp.dot(new, wn_ref[...],
                                 preferred_element_type=jnp.float32)

    args = list(aggs) + [cnt, own, wf, bf, wo1a, wo1b, bo1, wo2, bo2]
    specs = ([_tile_spec(tile, _EMB)] * nagg
             + [_tile_spec(tile, 1), _tile_spec(tile, _EMB)]
             + [_full_spec(a) for a in args[nagg + 2:]])
    if wnext is not None:
        args.append(wnext)
        specs.append(_full_spec(wnext))
    return pl.pallas_call(
        body,
        out_shape=jax.ShapeDtypeStruct((n, _EMB), jnp.float32),
        grid=(n // tile,),
        in_specs=specs,
        out_specs=_tile_spec(tile, _EMB),
        compiler_params=pltpu.CompilerParams(
            dimension_semantics=("parallel",)),
    )(*args)


def _pool_head_kernel(v_ref, starts_ref, ends_ref, recip_ref,
                      w1a_ref, b1a_ref, w1b_ref, b1b_ref,
                      w2ap_ref, w2ao_ref, b2a_ref,
                      w3ap_ref, w3ao_ref, b3a_ref,
                      w1bp_ref, w2bp_ref, w3bp_ref, bout_ref, o_ref):
    nvp = v_ref.shape[0]
    bsz = starts_ref.shape[0]
    r = jax.lax.broadcasted_iota(jnp.int32, (bsz, nvp), 1)
    inseg = (r >= starts_ref[...]) & (r < ends_ref[...])
    pool_w = jnp.where(inseg, recip_ref[...], 0.0)
    pred = jnp.dot(pool_w, v_ref[...], preferred_element_type=jnp.float32)
    tp = jnp.tanh(pred)
    h1 = jnp.tanh(jnp.dot(tp, w1a_ref[...],
                          preferred_element_type=jnp.float32) + b1a_ref[...])
    to1 = jnp.tanh(jnp.dot(h1, w1b_ref[...],
                           preferred_element_type=jnp.float32) + b1b_ref[...])
    h2 = jnp.tanh(jnp.dot(tp, w2ap_ref[...], preferred_element_type=jnp.float32)
                  + jnp.dot(to1, w2ao_ref[...], preferred_element_type=jnp.float32)
                  + b2a_ref[...])
    h3 = jnp.tanh(jnp.dot(tp, w3ap_ref[...], preferred_element_type=jnp.float32)
                  + jnp.dot(to1, w3ao_ref[...], preferred_element_type=jnp.float32)
                  + b3a_ref[...])
    o_ref[...] = (jnp.dot(h1, w1bp_ref[...], preferred_element_type=jnp.float32)
                  + jnp.dot(h2, w2bp_ref[...], preferred_element_type=jnp.float32)
                  + jnp.dot(h3, w3bp_ref[...], preferred_element_type=jnp.float32)
                  + bout_ref[...])


def _pool_and_head(v, starts_col, ends_col, recip_col, hp):
    bsz = starts_col.shape[0]
    args = (v, starts_col, ends_col, recip_col,
            hp['w1a'], hp['b1a'], hp['w1b'], hp['b1b'],
            hp['w2ap'], hp['w2ao'], hp['b2a'],
            hp['w3ap'], hp['w3ao'], hp['b3a'],
            hp['w1bp'], hp['w2bp'], hp['w3bp'], hp['bout'])
    vmem = pl.BlockSpec(memory_space=pltpu.MemorySpace.VMEM)
    return pl.pallas_call(
        _pool_head_kernel,
        out_shape=jax.ShapeDtypeStruct((bsz, _HEAD_W), jnp.float32),
        in_specs=[vmem] * len(args),
        out_specs=vmem,
    )(*args)


_EDGE_TILE = 1024


_ACC_SPLIT = 5
_WRITE_STEPS = 8


def _edge_conv_body(masked, nacc, nblk2, rows_blk):
    def body(*refs):
        if masked:
            tgt_ref, src_ref, ef_ref, val_ref, rt_ref, lt_ref, we_ref = refs[:7]
            rest = refs[7:]
        else:
            tgt_ref, src_ref, ef_ref, rt_ref, lt_ref, we_ref = refs[:6]
            rest = refs[6:]
        o_ref = rest[0]
        acc_refs = rest[1:]
        j = pl.program_id(1)

        @pl.when(j == 0)
        def _init():
            for a in acc_refs:
                a[...] = jnp.zeros(a.shape, jnp.float32)

        @pl.when(j < nblk2)
        def _accumulate():
            for mi in range(_EDGE_TILE):
                ti = tgt_ref[0, 0, mi]
                si = src_ref[0, 0, mi]
                e = ef_ref[0, 0, mi]
                val = jnp.tanh(rt_ref.at[ti][:] + lt_ref.at[si][:]
                               + e * we_ref[...])
                if masked:
                    val = val * val_ref[0, 0, mi]
                a = acc_refs[mi % nacc]
                a.at[ti][:] = a.at[ti][:] + val

        @pl.when(j >= nblk2)
        def _writeout():
            w = j - nblk2
            sl = pl.ds(w * rows_blk, rows_blk)
            tot = acc_refs[0][sl]
            for a in acc_refs[1:]:
                tot = tot + a[sl]
            o_ref[...] = tot
    return body


def _edge_messages(rt, lt, we_row, tgt_idx, src_idx, ef, valid, nseg):
    nep = tgt_idx.shape[0]
    nblk = nep // _EDGE_TILE
    ncores = 2 if nblk % 2 == 0 else 1
    nblk2 = nblk // ncores
    nw = _WRITE_STEPS if nseg % _WRITE_STEPS == 0 else 1
    rows_blk = nseg // nw
    tgt_b = tgt_idx.reshape(nblk, 1, _EDGE_TILE)
    src_b = src_idx.reshape(nblk, 1, _EDGE_TILE)
    ef_b = ef.reshape(nblk, 1, _EDGE_TILE)
    rt3 = rt.reshape(rt.shape[0], 1, _EMB)
    lt3 = lt.reshape(lt.shape[0], 1, _EMB)

    masked = valid is not None

    def idx_map(c, j):
        return (c * nblk2 + jnp.minimum(j, nblk2 - 1), 0, 0)

    idx_spec = pl.BlockSpec((1, 1, _EDGE_TILE), idx_map,
                            memory_space=pltpu.MemorySpace.SMEM)
    args = [tgt_b, src_b, ef_b]
    specs = [idx_spec, idx_spec, idx_spec]
    if masked:
        args.append(valid.reshape(nblk, 1, _EDGE_TILE))
        specs.append(idx_spec)
    args += [rt3, lt3, we_row]
    specs += [pl.BlockSpec(rt3.shape, lambda c, j: (0, 0, 0)),
              pl.BlockSpec(lt3.shape, lambda c, j: (0, 0, 0)),
              pl.BlockSpec(we_row.shape, lambda c, j: (0, 0))]

    def out_map(c, j):
        return (c * nw + jnp.clip(j - nblk2, 0, nw - 1), 0, 0)

    acc = pl.pallas_call(
        _edge_conv_body(masked, _ACC_SPLIT, nblk2, rows_blk),
        out_shape=jax.ShapeDtypeStruct((ncores * nseg, 1, _EMB),
                                       jnp.float32),
        grid=(ncores, nblk2 + nw),
        in_specs=specs,
        out_specs=pl.BlockSpec((rows_blk, 1, _EMB), out_map),
        scratch_shapes=[pltpu.VMEM((nseg, 1, _EMB), jnp.float32)
                        for _ in range(_ACC_SPLIT)],
        compiler_params=pltpu.CompilerParams(
            dimension_semantics=("parallel", "arbitrary"),
            flags={"XLA_TPU_STORE_TO_LOAD_FORWARDING_WINDOW": 8192},
            disable_bounds_checks=True),
    )(*args)

    a = acc.reshape(ncores, nseg, _EMB)
    return [a[c] for c in range(ncores)]


def kernel(cons_feat, edge_indices, edge_feat, var_feat, n_cons_per_sample,
           n_vars_per_sample, ce_w1, ce_b1, ce_w2, ce_b2, ve_w1, ve_b1, ve_w2,
           ve_b2, cvc_ws, cvc_bs, cvc_wf, cvc_bf, cvc_wo1, cvc_bo1, cvc_wo2,
           cvc_bo2, ccv_ws, ccv_bs, ccv_wf, ccv_bf, ccv_wo1, ccv_bo1, ccv_wo2,
           ccv_bo2, hd_w1a, hd_b1a, hd_w1b, hd_b1b, hd_w2ap, hd_w2ao, hd_b2a,
           hd_w3ap, hd_w3ao, hd_b3a, hd_w1bp, hd_w2bp, hd_w3bp, hd_bout):
    del n_cons_per_sample
    nc, nv, ne = cons_feat.shape[0], var_feat.shape[0], edge_feat.shape[0]
    bsz = n_vars_per_sample.shape[0]

    ncp = _ceil_to(max(nc, 1), _ROW_TILE)
    nvp = _ceil_to(max(nv, 1), _ROW_TILE)
    nep = _ceil_to(max(ne, 1), _EDGE_TILE)

    c_in = jnp.pad(cons_feat.astype(jnp.float32), ((0, ncp - nc), (0, 0)))
    v_in = jnp.pad(var_feat.astype(jnp.float32), ((0, nvp - nv), (0, 0)))
    ef = jnp.pad(edge_feat.astype(jnp.float32), ((0, nep - ne), (0, 0)))
    cidx = jnp.pad(edge_indices[0].astype(jnp.int32), (0, nep - ne))
    vidx = jnp.pad(edge_indices[1].astype(jnp.int32), (0, nep - ne))
    if nep == ne:
        valid = None
        ones = jnp.ones((nep, 1), jnp.float32)
    else:
        valid = (jnp.arange(nep) < ne).astype(jnp.float32)[:, None]
        ones = valid

    wl1, we1, wr1 = cvc_ws[:_EMB], cvc_ws[_EMB:_EMB + 1], cvc_ws[_EMB + 1:]
    wl2, we2, wr2 = ccv_ws[:_EMB], ccv_ws[_EMB:_EMB + 1], ccv_ws[_EMB + 1:]

    c_emb, rt1 = _embed_and_project(c_in, ce_w1, ce_b1, ce_w2, ce_b2,
                                    [(wl1, cvc_bs)])
    v_emb, lt1, rt2 = _embed_and_project(v_in, ve_w1, ve_b1, ve_w2, ve_b2,
                                         [(wr1, None), (wl2, ccv_bs)])

    cnt_c = jax.ops.segment_sum(ones, cidx, num_segments=ncp)
    cnt_v = jax.ops.segment_sum(ones, vidx, num_segments=nvp)

    aggs1 = _edge_messages(rt1, lt1, we1, cidx, vidx, ef, valid, ncp)
    lt2 = _conv_out_call(aggs1, cnt_c, c_emb, cvc_wf, cvc_bf,
                         cvc_wo1[:_EMB], cvc_wo1[_EMB:], cvc_bo1,
                         cvc_wo2, cvc_bo2, wnext=wr2)

    aggs2 = _edge_messages(rt2, lt2, we2, vidx, cidx, ef, valid, nvp)
    v2 = _conv_out_call(aggs2, cnt_v, v_emb, ccv_wf, ccv_bf,
                        ccv_wo1[:_EMB], ccv_wo1[_EMB:], ccv_bo1,
                        ccv_wo2, ccv_bo2, wnext=None)

    nvars = n_vars_per_sample.astype(jnp.int32)
    ends = jnp.cumsum(nvars)
    starts_col = (ends - nvars).reshape(bsz, 1)
    ends_col = ends.reshape(bsz, 1)
    recip_col = (1.0 / jnp.maximum(nvars, 1).astype(jnp.float32)).reshape(bsz, 1)
    hp = dict(w1a=hd_w1a, b1a=hd_b1a, w1b=hd_w1b, b1b=hd_b1b,
              w2ap=hd_w2ap, w2ao=hd_w2ao, b2a=hd_b2a,
              w3ap=hd_w3ap, w3ao=hd_w3ao, b3a=hd_b3a,
              w1bp=hd_w1bp, w2bp=hd_w2bp, w3bp=hd_w3bp, bout=hd_bout)
    out = _pool_and_head(v2, starts_col, ends_col, recip_col, hp)
    return out[:, :_OUT_COLS]


# speedup vs baseline: 1.7765x; 1.4388x over previous
import jax
import jax.numpy as jnp
from jax.experimental import pallas as pl
from jax.experimental.pallas import tpu as pltpu

_EMB = 64
_ROW_TILE = 512
_HEAD_W = 128
_OUT_COLS = 14 + 56 + 56


def _ceil_to(n, m):
    return ((n + m - 1) // m) * m


def _tile_spec(tile, cols):
    return pl.BlockSpec((tile, cols), lambda i: (i, 0))


def _full_spec(arr):
    return pl.BlockSpec(arr.shape, lambda i: (0,) * arr.ndim)


def _embed_and_project(x, w1, b1, w2, b2, projs, tile=_ROW_TILE):
    n = x.shape[0]
    nproj = len(projs)
    has_bias = [pb is not None for (_, pb) in projs]
    args = [x, w1, b1, w2, b2]
    specs = [_tile_spec(tile, x.shape[1]), _full_spec(w1), _full_spec(b1),
             _full_spec(w2), _full_spec(b2)]
    for (pw, pb) in projs:
        args.append(pw)
        specs.append(_full_spec(pw))
        if pb is not None:
            args.append(pb)
            specs.append(_full_spec(pb))

    def body(x_ref, w1_ref, b1_ref, w2_ref, b2_ref, *rest):
        nin = sum(1 + int(hb) for hb in has_bias)
        in_it = iter(rest[:nin])
        outs = rest[nin:]
        h = jnp.tanh(jnp.dot(x_ref[...], w1_ref[...],
                             preferred_element_type=jnp.float32) + b1_ref[...])
        emb = jnp.tanh(jnp.dot(h, w2_ref[...],
                               preferred_element_type=jnp.float32) + b2_ref[...])
        outs[0][...] = emb
        for k in range(nproj):
            wref = next(in_it)
            p = jnp.dot(emb, wref[...], preferred_element_type=jnp.float32)
            if has_bias[k]:
                p = p + next(in_it)[...]
            outs[1 + k][...] = p

    out_shapes = tuple(jax.ShapeDtypeStruct((n, _EMB), jnp.float32)
                       for _ in range(1 + nproj))
    out_specs = tuple(_tile_spec(tile, _EMB) for _ in range(1 + nproj))
    return pl.pallas_call(
        body,
        out_shape=out_shapes,
        grid=(n // tile,),
        in_specs=specs,
        out_specs=out_specs,
        compiler_params=pltpu.CompilerParams(
            dimension_semantics=("parallel",)),
    )(*args)


def _conv_out_call(aggs, cnt, own, wf, bf, wo1a, wo1b, bo1, wo2, bo2,
                   wnext=None, tile=_ROW_TILE):
    nagg = len(aggs)
    n = aggs[0].shape[0]

    def body(*refs):
        agg_refs = refs[:nagg]
        (cnt_ref, own_ref, wf_ref, bf_ref, wo1a_ref, wo1b_ref,
         bo1_ref, wo2_ref, bo2_ref) = refs[nagg:nagg + 9]
        rest = refs[nagg + 9:]
        araw = agg_refs[0][...]
        for k in range(1, nagg):
            araw = araw + agg_refs[k][...]
        a = jnp.dot(araw, wf_ref[...],
                    preferred_element_type=jnp.float32) + cnt_ref[...] * bf_ref[...]
        h = jnp.tanh(jnp.dot(a, wo1a_ref[...], preferred_element_type=jnp.float32)
                     + jnp.dot(own_ref[...], wo1b_ref[...],
                               preferred_element_type=jnp.float32)
                     + bo1_ref[...])
        new = jnp.dot(h, wo2_ref[...],
                      preferred_element_type=jnp.float32) + bo2_ref[...]
        if wnext is None:
            rest[-1][...] = new
        else:
            wn_ref, o_ref = rest
            o_ref[...] = jnp.dot(new, wn_ref[...],
                                 preferred_element_type=jnp.float32)

    args = list(aggs) + [cnt, own, wf, bf, wo1a, wo1b, bo1, wo2, bo2]
    specs = ([_tile_spec(tile, _EMB)] * nagg
             + [_tile_spec(tile, 1), _tile_spec(tile, _EMB)]
             + [_full_spec(a) for a in args[nagg + 2:]])
    if wnext is not None:
        args.append(wnext)
        specs.append(_full_spec(wnext))
    return pl.pallas_call(
        body,
        out_shape=jax.ShapeDtypeStruct((n, _EMB), jnp.float32),
        grid=(n // tile,),
        in_specs=specs,
        out_specs=_tile_spec(tile, _EMB),
        compiler_params=pltpu.CompilerParams(
            dimension_semantics=("parallel",)),
    )(*args)


def _pool_head_kernel(v_ref, starts_ref, ends_ref, recip_ref,
                      w1a_ref, b1a_ref, w1b_ref, b1b_ref,
                      w2ap_ref, w2ao_ref, b2a_ref,
                      w3ap_ref, w3ao_ref, b3a_ref,
                      w1bp_ref, w2bp_ref, w3bp_ref, bout_ref, o_ref):
    nvp = v_ref.shape[0]
    bsz = starts_ref.shape[0]
    r = jax.lax.broadcasted_iota(jnp.int32, (bsz, nvp), 1)
    inseg = (r >= starts_ref[...]) & (r < ends_ref[...])
    pool_w = jnp.where(inseg, recip_ref[...], 0.0)
    pred = jnp.dot(pool_w, v_ref[...], preferred_element_type=jnp.float32)
    tp = jnp.tanh(pred)
    h1 = jnp.tanh(jnp.dot(tp, w1a_ref[...],
                          preferred_element_type=jnp.float32) + b1a_ref[...])
    to1 = jnp.tanh(jnp.dot(h1, w1b_ref[...],
                           preferred_element_type=jnp.float32) + b1b_ref[...])
    h2 = jnp.tanh(jnp.dot(tp, w2ap_ref[...], preferred_element_type=jnp.float32)
                  + jnp.dot(to1, w2ao_ref[...], preferred_element_type=jnp.float32)
                  + b2a_ref[...])
    h3 = jnp.tanh(jnp.dot(tp, w3ap_ref[...], preferred_element_type=jnp.float32)
                  + jnp.dot(to1, w3ao_ref[...], preferred_element_type=jnp.float32)
                  + b3a_ref[...])
    o_ref[...] = (jnp.dot(h1, w1bp_ref[...], preferred_element_type=jnp.float32)
                  + jnp.dot(h2, w2bp_ref[...], preferred_element_type=jnp.float32)
                  + jnp.dot(h3, w3bp_ref[...], preferred_element_type=jnp.float32)
                  + bout_ref[...])


def _pool_and_head(v, starts_col, ends_col, recip_col, hp):
    bsz = starts_col.shape[0]
    args = (v, starts_col, ends_col, recip_col,
            hp['w1a'], hp['b1a'], hp['w1b'], hp['b1b'],
            hp['w2ap'], hp['w2ao'], hp['b2a'],
            hp['w3ap'], hp['w3ao'], hp['b3a'],
            hp['w1bp'], hp['w2bp'], hp['w3bp'], hp['bout'])
    vmem = pl.BlockSpec(memory_space=pltpu.MemorySpace.VMEM)
    return pl.pallas_call(
        _pool_head_kernel,
        out_shape=jax.ShapeDtypeStruct((bsz, _HEAD_W), jnp.float32),
        in_specs=[vmem] * len(args),
        out_specs=vmem,
    )(*args)


_EDGE_TILE = 1024


_ACC_SPLIT = 5
_WRITE_STEPS = 8


def _edge_conv_body(masked, nacc, nblk2, rows_blk):
    def body(*refs):
        if masked:
            tgt_ref, src_ref, ef_ref, val_ref, rt_ref, lt_ref, we_ref = refs[:7]
            rest = refs[7:]
        else:
            tgt_ref, src_ref, ef_ref, rt_ref, lt_ref, we_ref = refs[:6]
            rest = refs[6:]
        o_ref = rest[0]
        acc_refs = rest[1:]
        j = pl.program_id(1)

        @pl.when(j == 0)
        def _init():
            for a in acc_refs:
                a[...] = jnp.zeros(a.shape, jnp.float32)

        @pl.when(j < nblk2)
        def _accumulate():
            for mi in range(_EDGE_TILE):
                ti = tgt_ref[0, 0, mi]
                si = src_ref[0, 0, mi]
                e = ef_ref[0, 0, mi]
                val = jnp.tanh(rt_ref.at[ti][:] + lt_ref.at[si][:]
                               + e * we_ref[...])
                if masked:
                    val = val * val_ref[0, 0, mi]
                a = acc_refs[mi % nacc]
                a.at[ti][:] = a.at[ti][:] + val

        @pl.when(j >= nblk2)
        def _writeout():
            w = j - nblk2
            sl = pl.ds(w * rows_blk, rows_blk)
            tot = acc_refs[0][sl]
            for a in acc_refs[1:]:
                tot = tot + a[sl]
            o_ref[...] = tot
    return body


def _edge_messages(rt, lt, we_row, tgt_idx, src_idx, ef, valid, nseg):
    nep = tgt_idx.shape[0]
    nblk = nep // _EDGE_TILE
    ncores = 2 if nblk % 2 == 0 else 1
    nblk2 = nblk // ncores
    nw = _WRITE_STEPS if nseg % _WRITE_STEPS == 0 else 1
    rows_blk = nseg // nw
    tgt_b = tgt_idx.reshape(nblk, 1, _EDGE_TILE)
    src_b = src_idx.reshape(nblk, 1, _EDGE_TILE)
    ef_b = ef.reshape(nblk, 1, _EDGE_TILE)
    rt3 = jnp.pad(rt, ((0, 0), (0, _EMB)),
                  constant_values=10.0).reshape(rt.shape[0], 1, 2 * _EMB)
    lt3 = jnp.pad(lt, ((0, 0), (0, _EMB)),
                  constant_values=10.0).reshape(lt.shape[0], 1, 2 * _EMB)
    we_row = jnp.pad(we_row, ((0, 0), (0, _EMB)))

    masked = valid is not None

    def idx_map(c, j):
        return (c * nblk2 + jnp.minimum(j, nblk2 - 1), 0, 0)

    idx_spec = pl.BlockSpec((1, 1, _EDGE_TILE), idx_map,
                            memory_space=pltpu.MemorySpace.SMEM)
    args = [tgt_b, src_b, ef_b]
    specs = [idx_spec, idx_spec, idx_spec]
    if masked:
        args.append(valid.reshape(nblk, 1, _EDGE_TILE))
        specs.append(idx_spec)
    args += [rt3, lt3, we_row]
    specs += [pl.BlockSpec(rt3.shape, lambda c, j: (0, 0, 0)),
              pl.BlockSpec(lt3.shape, lambda c, j: (0, 0, 0)),
              pl.BlockSpec(we_row.shape, lambda c, j: (0, 0))]

    def out_map(c, j):
        return (c * nw + jnp.clip(j - nblk2, 0, nw - 1), 0, 0)

    acc = pl.pallas_call(
        _edge_conv_body(masked, _ACC_SPLIT, nblk2, rows_blk),
        out_shape=jax.ShapeDtypeStruct((ncores * nseg, 1, 2 * _EMB),
                                       jnp.float32),
        grid=(ncores, nblk2 + nw),
        in_specs=specs,
        out_specs=pl.BlockSpec((rows_blk, 1, 2 * _EMB), out_map),
        scratch_shapes=[pltpu.VMEM((nseg, 1, 2 * _EMB), jnp.float32)
                        for _ in range(_ACC_SPLIT)],
        compiler_params=pltpu.CompilerParams(
            dimension_semantics=("parallel", "arbitrary"),
            flags={"XLA_TPU_STORE_TO_LOAD_FORWARDING_WINDOW": 8192},
            disable_bounds_checks=True),
    )(*args)

    a = acc.reshape(ncores, nseg, 2 * _EMB)
    parts = [a[c, :, :_EMB] for c in range(ncores)]
    cnt = a[0, :, _EMB:_EMB + 1]
    for c in range(1, ncores):
        cnt = cnt + a[c, :, _EMB:_EMB + 1]
    return parts, cnt


def kernel(cons_feat, edge_indices, edge_feat, var_feat, n_cons_per_sample,
           n_vars_per_sample, ce_w1, ce_b1, ce_w2, ce_b2, ve_w1, ve_b1, ve_w2,
           ve_b2, cvc_ws, cvc_bs, cvc_wf, cvc_bf, cvc_wo1, cvc_bo1, cvc_wo2,
           cvc_bo2, ccv_ws, ccv_bs, ccv_wf, ccv_bf, ccv_wo1, ccv_bo1, ccv_wo2,
           ccv_bo2, hd_w1a, hd_b1a, hd_w1b, hd_b1b, hd_w2ap, hd_w2ao, hd_b2a,
           hd_w3ap, hd_w3ao, hd_b3a, hd_w1bp, hd_w2bp, hd_w3bp, hd_bout):
    del n_cons_per_sample
    nc, nv, ne = cons_feat.shape[0], var_feat.shape[0], edge_feat.shape[0]
    bsz = n_vars_per_sample.shape[0]

    ncp = _ceil_to(max(nc, 1), _ROW_TILE)
    nvp = _ceil_to(max(nv, 1), _ROW_TILE)
    nep = _ceil_to(max(ne, 1), _EDGE_TILE)

    c_in = jnp.pad(cons_feat.astype(jnp.float32), ((0, ncp - nc), (0, 0)))
    v_in = jnp.pad(var_feat.astype(jnp.float32), ((0, nvp - nv), (0, 0)))
    ef = jnp.pad(edge_feat.astype(jnp.float32), ((0, nep - ne), (0, 0)))
    cidx = jnp.pad(edge_indices[0].astype(jnp.int32), (0, nep - ne))
    vidx = jnp.pad(edge_indices[1].astype(jnp.int32), (0, nep - ne))
    if nep == ne:
        valid = None
    else:
        valid = (jnp.arange(nep) < ne).astype(jnp.float32)[:, None]

    wl1, we1, wr1 = cvc_ws[:_EMB], cvc_ws[_EMB:_EMB + 1], cvc_ws[_EMB + 1:]
    wl2, we2, wr2 = ccv_ws[:_EMB], ccv_ws[_EMB:_EMB + 1], ccv_ws[_EMB + 1:]

    c_emb, rt1 = _embed_and_project(c_in, ce_w1, ce_b1, ce_w2, ce_b2,
                                    [(wl1, cvc_bs)])
    v_emb, lt1, rt2 = _embed_and_project(v_in, ve_w1, ve_b1, ve_w2, ve_b2,
                                         [(wr1, None), (wl2, ccv_bs)])

    aggs1, cnt_c = _edge_messages(rt1, lt1, we1, cidx, vidx, ef, valid, ncp)
    lt2 = _conv_out_call(aggs1, cnt_c, c_emb, cvc_wf, cvc_bf,
                         cvc_wo1[:_EMB], cvc_wo1[_EMB:], cvc_bo1,
                         cvc_wo2, cvc_bo2, wnext=wr2)

    aggs2, cnt_v = _edge_messages(rt2, lt2, we2, vidx, cidx, ef, valid, nvp)
    v2 = _conv_out_call(aggs2, cnt_v, v_emb, ccv_wf, ccv_bf,
                        ccv_wo1[:_EMB], ccv_wo1[_EMB:], ccv_bo1,
                        ccv_wo2, ccv_bo2, wnext=None)

    nvars = n_vars_per_sample.astype(jnp.int32)
    ends = jnp.cumsum(nvars)
    starts_col = (ends - nvars).reshape(bsz, 1)
    ends_col = ends.reshape(bsz, 1)
    recip_col = (1.0 / jnp.maximum(nvars, 1).astype(jnp.float32)).reshape(bsz, 1)
    hp = dict(w1a=hd_w1a, b1a=hd_b1a, w1b=hd_w1b, b1b=hd_b1b,
              w2ap=hd_w2ap, w2ao=hd_w2ao, b2a=hd_b2a,
              w3ap=hd_w3ap, w3ao=hd_w3ao, b3a=hd_b3a,
              w1bp=hd_w1bp, w2bp=hd_w2bp, w3bp=hd_w3bp, bout=hd_bout)
    out = _pool_and_head(v2, starts_col, ends_col, recip_col, hp)
    return out[:, :_OUT_COLS]
